# baseline (speedup 1.0000x reference)
import jax
import jax.numpy as jnp
from jax.experimental import pallas as pl
from jax.experimental.pallas import tpu as pltpu

_BM = 512
_VMEM_LIMIT = 64 * 1024 * 1024


def _matmul_bias_kernel(x_ref, w_ref, b_ref, o_ref):
    o_ref[...] = (
        jax.lax.dot_general(
            x_ref[...],
            w_ref[...],
            dimension_numbers=(((1,), (1,)), ((), ())),
            preferred_element_type=jnp.float32,
        )
        + b_ref[...]
    )


@jax.jit
def kernel(x, weight, bias):
    B, D_in = x.shape
    D_out = weight.shape[0]

    x_bf = x.astype(jnp.bfloat16)
    w_bf = weight.astype(jnp.bfloat16)
    b2 = bias.reshape(1, D_out)

    return pl.pallas_call(
        _matmul_bias_kernel,
        out_shape=jax.ShapeDtypeStruct((B, D_out), jnp.float32),
        grid=(B // _BM,),
        in_specs=[
            pl.BlockSpec((_BM, D_in), lambda i: (i, 0)),
            pl.BlockSpec((D_out, D_in), lambda i: (0, 0)),
            pl.BlockSpec((1, D_out), lambda i: (0, 0)),
        ],
        out_specs=pl.BlockSpec((_BM, D_out), lambda i: (i, 0)),
        compiler_params=pltpu.CompilerParams(
            dimension_semantics=("parallel",),
            vmem_limit_bytes=_VMEM_LIMIT,
        ),
    )(x_bf, w_bf, b2)


# speedup vs baseline: 1.1728x; 1.1728x over previous
import jax
import jax.numpy as jnp
from jax.experimental import pallas as pl
from jax.experimental.pallas import tpu as pltpu

_BM = 1024
_BN = 1024
_VMEM_LIMIT = 64 * 1024 * 1024


def _matmul_bias_kernel(x_ref, w_ref, b_ref, o_ref):
    o_ref[...] = (
        jax.lax.dot_general(
            x_ref[...].astype(jnp.bfloat16),
            w_ref[...],
            dimension_numbers=(((1,), (1,)), ((), ())),
            preferred_element_type=jnp.float32,
        )
        + b_ref[...]
    )


@jax.jit
def kernel(x, weight, bias):
    B, D_in = x.shape
    D_out = weight.shape[0]

    w_bf = weight.astype(jnp.bfloat16)
    b2 = bias.reshape(1, D_out)

    m_grid = B // _BM
    n_grid = D_out // _BN

    return pl.pallas_call(
        _matmul_bias_kernel,
        out_shape=jax.ShapeDtypeStruct((B, D_out), jnp.float32),
        grid=(m_grid, n_grid),
        in_specs=[
            pl.BlockSpec((_BM, D_in), lambda i, j: (i, 0)),
            pl.BlockSpec((_BN, D_in), lambda i, j: (j, 0)),
            pl.BlockSpec((1, _BN), lambda i, j: (0, j)),
        ],
        out_specs=pl.BlockSpec((_BM, _BN), lambda i, j: (i, j)),
        compiler_params=pltpu.CompilerParams(
            dimension_semantics=("parallel", "parallel"),
            vmem_limit_bytes=_VMEM_LIMIT,
        ),
    )(x, w_bf, b2)


# speedup vs baseline: 1.2768x; 1.0887x over previous
import jax
import jax.numpy as jnp
from jax.experimental import pallas as pl
from jax.experimental.pallas import tpu as pltpu

_BM = 1024
_BK = 1024
_NH = 2048
_VMEM_LIMIT = 64 * 1024 * 1024


def _matmul_bias_kernel(x_ref, w_ref, b_ref, o_ref, wbf_ref):
    i = pl.program_id(1)
    k = pl.program_id(2)

    @pl.when(i == 0)
    def _():
        wbf_ref[:, pl.ds(k * _BK, _BK)] = w_ref[...].astype(jnp.bfloat16)

    @pl.when(k == 0)
    def _():
        o_ref[...] = jnp.broadcast_to(b_ref[...], o_ref.shape)

    o_ref[...] += jax.lax.dot_general(
        x_ref[...].astype(jnp.bfloat16),
        wbf_ref[:, pl.ds(k * _BK, _BK)],
        dimension_numbers=(((1,), (1,)), ((), ())),
        preferred_element_type=jnp.float32,
    )


@jax.jit
def kernel(x, weight, bias):
    B, D_in = x.shape
    D_out = weight.shape[0]

    b2 = bias.reshape(1, D_out)
    n_grid = D_out // _NH
    m_grid = B // _BM
    k_grid = D_in // _BK
    last_k = k_grid - 1

    return pl.pallas_call(
        _matmul_bias_kernel,
        out_shape=jax.ShapeDtypeStruct((B, D_out), jnp.float32),
        grid=(n_grid, m_grid, k_grid),
        in_specs=[
            pl.BlockSpec((_BM, _BK), lambda n, i, k: (i, k)),
            pl.BlockSpec(
                (_NH, _BK),
                lambda n, i, k: (n, jax.lax.select(i == 0, k, last_k)),
            ),
            pl.BlockSpec((1, _NH), lambda n, i, k: (0, n)),
        ],
        out_specs=pl.BlockSpec((_BM, _NH), lambda n, i, k: (i, n)),
        scratch_shapes=[pltpu.VMEM((_NH, D_in), jnp.bfloat16)],
        compiler_params=pltpu.CompilerParams(
            dimension_semantics=("parallel", "arbitrary", "arbitrary"),
            vmem_limit_bytes=_VMEM_LIMIT,
        ),
    )(x, weight, b2)
